# baseline (speedup 1.0000x reference)
import jax
import jax.numpy as jnp
from jax import lax
from jax.experimental import pallas as pl
from jax.experimental.pallas import tpu as pltpu
from jax.experimental.pallas import tpu_sc as plsc

C = 16
K = 27
HP = 65536
H = 65536

NC = 2
NS = 16
CG = C // NC
N = K * HP
N_TILE = N // NS
CHUNK = 4096
STEPS = N_TILE // CHUNK
ZB = H // NS
TB = 4096


def _tc_transpose_body(x_ref, y_ref):
    y_ref[...] = x_ref[...].T


def _body(data_hbm, oct_hbm, zero_hbm, out_hbm, idx_v, val_v,
          isem0, isem1, vsem0, vsem1, ssem0, ssem1, acc):
    core = lax.axis_index("c")
    sub = lax.axis_index("s")
    isem = (isem0, isem1)
    vsem = (vsem0, vsem1)
    ssem = (ssem0, ssem1)

    pltpu.sync_copy(zero_hbm, acc.at[pl.ds(sub * ZB, ZB), :])
    plsc.subcore_barrier()

    base = sub * N_TILE

    def load(j, b):
        off = base + j * CHUNK
        d1 = pltpu.async_copy(oct_hbm.at[pl.ds(off, CHUNK)],
                              idx_v.at[b], isem[b])
        d2 = pltpu.async_copy(
            data_hbm.at[pl.ds(off, CHUNK), pl.ds(core * CG, CG)],
            val_v.at[b], vsem[b])
        return d1, d2

    loads = {0: load(0, 0)}
    scats = {}
    for j in range(STEPS):
        b = j & 1
        d1, d2 = loads.pop(j)
        d1.wait()
        d2.wait()
        scats[j] = pltpu.async_copy(val_v.at[b], acc.at[idx_v.at[b]],
                                    ssem[b], add=True)
        if j + 1 < STEPS:
            if j >= 1:
                scats.pop(j - 1).wait()
            loads[j + 1] = load(j + 1, b ^ 1)
    for j in sorted(scats):
        scats[j].wait()
    plsc.subcore_barrier()

    start = sub * ZB
    pltpu.sync_copy(acc.at[pl.ds(start, ZB), :],
                    out_hbm.at[core, pl.ds(start, ZB), :])


@jax.jit
def kernel(data_in, octree):
    data_t = pl.pallas_call(
        _tc_transpose_body,
        grid=(N // TB,),
        in_specs=[pl.BlockSpec((C, TB), lambda i: (0, i))],
        out_specs=pl.BlockSpec((TB, C), lambda i: (i, 0)),
        out_shape=jax.ShapeDtypeStruct((N, C), jnp.float32),
    )(data_in.reshape(C, N))

    oct_flat = octree.T.reshape(N)
    zeros = jnp.zeros((ZB, CG), jnp.float32)

    mesh = plsc.VectorSubcoreMesh(core_axis_name="c", subcore_axis_name="s")
    scatter = pl.kernel(
        _body,
        out_type=jax.ShapeDtypeStruct((NC, H, CG), jnp.float32),
        mesh=mesh,
        compiler_params=pltpu.CompilerParams(use_tc_tiling_on_sc=False),
        scratch_types=(
            pltpu.VMEM((2, CHUNK), jnp.int32),
            pltpu.VMEM((2, CHUNK, CG), jnp.float32),
            pltpu.SemaphoreType.DMA,
            pltpu.SemaphoreType.DMA,
            pltpu.SemaphoreType.DMA,
            pltpu.SemaphoreType.DMA,
            pltpu.SemaphoreType.DMA,
            pltpu.SemaphoreType.DMA,
            pltpu.VMEM_SHARED((H, CG), jnp.float32),
        ),
    )
    out2 = scatter(data_t, oct_flat, zeros)
    return out2.transpose(0, 2, 1).reshape(C, H)


# speedup vs baseline: 1.5319x; 1.5319x over previous
import jax
import jax.numpy as jnp
from jax import lax
from jax.experimental import pallas as pl
from jax.experimental.pallas import tpu as pltpu
from jax.experimental.pallas import tpu_sc as plsc

C = 16
K = 27
HP = 65536
H = 65536

NC = 2
NS = 16
CG = C // NC
N = K * HP
N_TILE = N // NS
CHUNK = 4096
STEPS = N_TILE // CHUNK
ZB = H // NS
TB = 4096


def _body(data_hbm, oct_hbm, zero_hbm, out_hbm, idx_v, val_v,
          isem0, isem1, vsem0, vsem1, ssem0, ssem1, acc):
    core = lax.axis_index("c")
    sub = lax.axis_index("s")
    isem = (isem0, isem1)
    vsem = (vsem0, vsem1)
    ssem = (ssem0, ssem1)

    pltpu.sync_copy(zero_hbm, acc.at[pl.ds(sub * ZB, ZB), :])
    plsc.subcore_barrier()

    base = sub * N_TILE

    def load(j, b):
        off = base + j * CHUNK
        d1 = pltpu.async_copy(oct_hbm.at[pl.ds(off, CHUNK)],
                              idx_v.at[b], isem[b])
        d2 = pltpu.async_copy(
            data_hbm.at[pl.ds(off, CHUNK), pl.ds(core * CG, CG)],
            val_v.at[b], vsem[b])
        return d1, d2

    loads = {0: load(0, 0)}
    scats = {}
    for j in range(STEPS):
        b = j & 1
        d1, d2 = loads.pop(j)
        d1.wait()
        d2.wait()
        scats[j] = pltpu.async_copy(val_v.at[b], acc.at[idx_v.at[b]],
                                    ssem[b], add=True)
        if j + 1 < STEPS:
            if j >= 1:
                scats.pop(j - 1).wait()
            loads[j + 1] = load(j + 1, b ^ 1)
    for j in sorted(scats):
        scats[j].wait()
    plsc.subcore_barrier()

    start = sub * ZB
    pltpu.sync_copy(acc.at[pl.ds(start, ZB), :],
                    out_hbm.at[core, pl.ds(start, ZB), :])


@jax.jit
def kernel(data_in, octree):
    data_t = jnp.moveaxis(data_in, 0, -1).reshape(N, C)

    oct_flat = octree.T.reshape(N)
    zeros = jnp.zeros((ZB, CG), jnp.float32)

    mesh = plsc.VectorSubcoreMesh(core_axis_name="c", subcore_axis_name="s")
    scatter = pl.kernel(
        _body,
        out_type=jax.ShapeDtypeStruct((NC, H, CG), jnp.float32),
        mesh=mesh,
        compiler_params=pltpu.CompilerParams(use_tc_tiling_on_sc=False),
        scratch_types=(
            pltpu.VMEM((2, CHUNK), jnp.int32),
            pltpu.VMEM((2, CHUNK, CG), jnp.float32),
            pltpu.SemaphoreType.DMA,
            pltpu.SemaphoreType.DMA,
            pltpu.SemaphoreType.DMA,
            pltpu.SemaphoreType.DMA,
            pltpu.SemaphoreType.DMA,
            pltpu.SemaphoreType.DMA,
            pltpu.VMEM_SHARED((H, CG), jnp.float32),
        ),
    )
    out2 = scatter(data_t, oct_flat, zeros)
    return out2.transpose(0, 2, 1).reshape(C, H)
